# baseline (speedup 1.0000x reference)
import functools

import jax
import jax.numpy as jnp
from jax.experimental import pallas as pl
from jax.experimental.pallas import tpu as pltpu

N = 4096
B = 8
UNITS = 16
M = 3
CPAD = 32
F = B * CPAD
BLK = 512
NJ = N // BLK


def _build_body(a_ref, at_ref, amax_ref, dis_ref, acc_ref):
    j = pl.program_id(1)
    m = jnp.maximum(a_ref[...], at_ref[...].T)
    amax_ref[...] = m.astype(jnp.bfloat16)

    @pl.when(j == 0)
    def _():
        acc_ref[...] = jnp.zeros_like(acc_ref)

    acc_ref[...] += jnp.sum(m, axis=1, keepdims=True)

    @pl.when(j == NJ - 1)
    def _():
        d = acc_ref[...]
        dis_ref[...] = jnp.where(
            d > 0, jax.lax.rsqrt(jnp.maximum(d, 1e-12)), 0.0)


def _build(adj):
    return pl.pallas_call(
        _build_body,
        grid=(NJ, NJ),
        in_specs=[
            pl.BlockSpec((BLK, BLK), lambda i, j: (i, j)),
            pl.BlockSpec((BLK, BLK), lambda i, j: (j, i)),
        ],
        out_specs=[
            pl.BlockSpec((BLK, BLK), lambda i, j: (i, j)),
            pl.BlockSpec((BLK, 1), lambda i, j: (i, 0)),
        ],
        out_shape=[
            jax.ShapeDtypeStruct((N, N), jnp.bfloat16),
            jax.ShapeDtypeStruct((N, 1), jnp.float32),
        ],
        scratch_shapes=[pltpu.VMEM((BLK, 1), jnp.float32)],
    )(adj, adj)


def _sapply_body(c1, c2, a_ref, x_ref, disj_ref, disi_ref, z_ref, out_ref):
    j = pl.program_id(0)
    xs = (x_ref[...] * disj_ref[...]).astype(jnp.bfloat16)
    part = jnp.dot(a_ref[...], xs, preferred_element_type=jnp.float32)

    @pl.when(j == 0)
    def _():
        out_ref[...] = jnp.zeros_like(out_ref)

    out_ref[...] += part

    @pl.when(j == NJ - 1)
    def _():
        out_ref[...] = c1 * disi_ref[...] * out_ref[...] + c2 * z_ref[...]


def _sapply(amax, x, dis, z, c1, c2):
    return pl.pallas_call(
        functools.partial(_sapply_body, c1, c2),
        grid=(NJ,),
        in_specs=[
            pl.BlockSpec((N, BLK), lambda j: (0, j)),
            pl.BlockSpec((BLK, F), lambda j: (j, 0)),
            pl.BlockSpec((BLK, 1), lambda j: (j, 0)),
            pl.BlockSpec((N, 1), lambda j: (0, 0)),
            pl.BlockSpec((N, F), lambda j: (0, 0)),
        ],
        out_specs=pl.BlockSpec((N, F), lambda j: (0, 0)),
        out_shape=jax.ShapeDtypeStruct((N, F), jnp.float32),
    )(amax, x, dis, dis, z)


ROWS = N * B
RBLK = 4096
NR = ROWS // RBLK


def _gate_body(x0_ref, x1_ref, x2_ref, wr_ref, wu_ref, br_ref, bu_ref,
               hx_ref, rh_ref, u_ref):
    accr = br_ref[...]
    accu = bu_ref[...]
    for m, xr in enumerate((x0_ref, x1_ref, x2_ref)):
        x = xr[...]
        accr = accr + jnp.dot(x, wr_ref[m], preferred_element_type=jnp.float32)
        accu = accu + jnp.dot(x, wu_ref[m], preferred_element_type=jnp.float32)
    r = jax.nn.sigmoid(accr)
    u = jax.nn.sigmoid(accu)
    rh_ref[...] = r * hx_ref[...]
    u_ref[...] = u


def _gate(x0, x1, x2, wr, wu, br, bu, hx):
    return pl.pallas_call(
        _gate_body,
        grid=(NR,),
        in_specs=[
            pl.BlockSpec((RBLK, CPAD), lambda i: (i, 0)),
            pl.BlockSpec((RBLK, CPAD), lambda i: (i, 0)),
            pl.BlockSpec((RBLK, CPAD), lambda i: (i, 0)),
            pl.BlockSpec((M, CPAD, UNITS), lambda i: (0, 0, 0)),
            pl.BlockSpec((M, CPAD, UNITS), lambda i: (0, 0, 0)),
            pl.BlockSpec((1, UNITS), lambda i: (0, 0)),
            pl.BlockSpec((1, UNITS), lambda i: (0, 0)),
            pl.BlockSpec((RBLK, UNITS), lambda i: (i, 0)),
        ],
        out_specs=[
            pl.BlockSpec((RBLK, UNITS), lambda i: (i, 0)),
            pl.BlockSpec((RBLK, UNITS), lambda i: (i, 0)),
        ],
        out_shape=[
            jax.ShapeDtypeStruct((ROWS, UNITS), jnp.float32),
            jax.ShapeDtypeStruct((ROWS, UNITS), jnp.float32),
        ],
    )(x0, x1, x2, wr, wu, br, bu, hx)


def _cand_body(x0_ref, x1_ref, x2_ref, wc_ref, bc_ref, u_ref, hx_ref,
               nh_ref):
    acc = bc_ref[...]
    for m, xr in enumerate((x0_ref, x1_ref, x2_ref)):
        acc = acc + jnp.dot(xr[...], wc_ref[m],
                            preferred_element_type=jnp.float32)
    c = jnp.tanh(acc)
    u = u_ref[...]
    nh_ref[...] = u * hx_ref[...] + (1.0 - u) * c


def _cand(x0, x1, x2, wc, bc, u, hx):
    return pl.pallas_call(
        _cand_body,
        grid=(NR,),
        in_specs=[
            pl.BlockSpec((RBLK, CPAD), lambda i: (i, 0)),
            pl.BlockSpec((RBLK, CPAD), lambda i: (i, 0)),
            pl.BlockSpec((RBLK, CPAD), lambda i: (i, 0)),
            pl.BlockSpec((M, CPAD, UNITS), lambda i: (0, 0, 0)),
            pl.BlockSpec((1, UNITS), lambda i: (0, 0)),
            pl.BlockSpec((RBLK, UNITS), lambda i: (i, 0)),
            pl.BlockSpec((RBLK, UNITS), lambda i: (i, 0)),
        ],
        out_specs=pl.BlockSpec((RBLK, UNITS), lambda i: (i, 0)),
        out_shape=jax.ShapeDtypeStruct((ROWS, UNITS), jnp.float32),
    )(x0, x1, x2, wc, bc, u, hx)


def _prep_w(W, C, O):
    Wr = jnp.transpose(W.reshape(C, M, O), (1, 0, 2))
    return jnp.pad(Wr, ((0, 0), (0, CPAD - C), (0, 0)))


def _gconv_terms(amax, dis, x0):
    x1 = _sapply(amax, x0, dis, x0, -1.0, 0.0)
    x2 = _sapply(amax, x1, dis, x0, -2.0, -1.0)
    v = lambda t: t.reshape(ROWS, CPAD)
    return v(x0), v(x1), v(x2)


def kernel(inputs, hidden_state, adj, W0_gate, b0_gate, W0_cand, b0_cand,
           W1_gate, b1_gate, W1_cand, b1_cand):
    amax, dis = _build(adj)

    xin = inputs.T.reshape(N, B, 1)
    params = [(W0_gate, b0_gate, W0_cand, b0_cand, 1 + UNITS),
              (W1_gate, b1_gate, W1_cand, b1_cand, 2 * UNITS)]
    hs = []
    cur = xin
    for l in range(2):
        Wg, bg, Wc, bc, C = params[l]
        wg = _prep_w(Wg, C, 2 * UNITS)
        wr, wu = wg[:, :, :UNITS], wg[:, :, UNITS:]
        br, bu = bg[:UNITS].reshape(1, UNITS), bg[UNITS:].reshape(1, UNITS)
        wc = _prep_w(Wc, C, UNITS)
        bcv = bc.reshape(1, UNITS)

        hx = jnp.transpose(hidden_state[l].reshape(B, N, UNITS), (1, 0, 2))
        hxr = hx.reshape(ROWS, UNITS)

        cin = cur.shape[2]
        pad = CPAD - cin - UNITS
        x0g = jnp.concatenate(
            [cur, hx, jnp.zeros((N, B, pad), jnp.float32)], axis=2
        ).reshape(N, F)
        t0, t1, t2 = _gconv_terms(amax, dis, x0g)
        rh, u = _gate(t0, t1, t2, wr, wu, br, bu, hxr)

        x0c = jnp.concatenate(
            [cur, rh.reshape(N, B, UNITS),
             jnp.zeros((N, B, pad), jnp.float32)], axis=2
        ).reshape(N, F)
        c0, c1t, c2t = _gconv_terms(amax, dis, x0c)
        nh = _cand(c0, c1t, c2t, wc, bcv, u, hxr)

        nh3 = nh.reshape(N, B, UNITS)
        hs.append(jnp.transpose(nh3, (1, 0, 2)).reshape(B, N * UNITS))
        cur = nh3

    return hs[-1], jnp.stack(hs, axis=0)


# speedup vs baseline: 1.6497x; 1.6497x over previous
import functools

import jax
import jax.numpy as jnp
from jax.experimental import pallas as pl
from jax.experimental.pallas import tpu as pltpu

N = 4096
B = 8
UNITS = 16
M = 3
CPAD = 32
F = B * CPAD
BLK = 512
NJB = N // BLK
KBLK = 1024
NJ = N // KBLK
RBLK = 512
NRB = N // RBLK


def _build_body(a_ref, at_ref, amax_ref, dis_ref, acc_ref):
    j = pl.program_id(1)
    m = jnp.maximum(a_ref[...], at_ref[...].T)
    amax_ref[...] = m.astype(jnp.bfloat16)

    @pl.when(j == 0)
    def _():
        acc_ref[...] = jnp.zeros_like(acc_ref)

    acc_ref[...] += jnp.sum(m, axis=1, keepdims=True)

    @pl.when(j == NJB - 1)
    def _():
        d = acc_ref[...]
        dis_ref[...] = jnp.where(
            d > 0, jax.lax.rsqrt(jnp.maximum(d, 1e-12)), 0.0)


def _build(adj):
    return pl.pallas_call(
        _build_body,
        grid=(NJB, NJB),
        in_specs=[
            pl.BlockSpec((BLK, BLK), lambda i, j: (i, j)),
            pl.BlockSpec((BLK, BLK), lambda i, j: (j, i)),
        ],
        out_specs=[
            pl.BlockSpec((BLK, BLK), lambda i, j: (i, j)),
            pl.BlockSpec((BLK, 1), lambda i, j: (i, 0)),
        ],
        out_shape=[
            jax.ShapeDtypeStruct((N, N), jnp.bfloat16),
            jax.ShapeDtypeStruct((N, 1), jnp.float32),
        ],
        scratch_shapes=[pltpu.VMEM((BLK, 1), jnp.float32)],
    )(adj, adj)


def _sapply1_body(ci, a_ref, inp_ref, h_ref, disj_ref, disi_ref,
                  x0_ref, x1_ref):
    j = pl.program_id(0)
    pad = CPAD - ci - UNITS
    pieces = []
    for b in range(B):
        sub = [inp_ref[b], h_ref[b]]
        if pad:
            sub.append(jnp.zeros((KBLK, pad), jnp.float32))
        pieces.append(jnp.concatenate(sub, axis=1))
    x0v = jnp.concatenate(pieces, axis=1)
    x0_ref[...] = x0v
    xs = (x0v * disj_ref[...]).astype(jnp.bfloat16)
    part = jnp.dot(a_ref[...], xs, preferred_element_type=jnp.float32)

    @pl.when(j == 0)
    def _():
        x1_ref[...] = part

    @pl.when(j > 0)
    def _():
        x1_ref[...] += part

    @pl.when(j == NJ - 1)
    def _():
        x1_ref[...] = -disi_ref[...] * x1_ref[...]


def _sapply1(amax, dis, inp_b, h_b, ci):
    return pl.pallas_call(
        functools.partial(_sapply1_body, ci),
        grid=(NJ,),
        in_specs=[
            pl.BlockSpec((N, KBLK), lambda j: (0, j)),
            pl.BlockSpec((B, KBLK, ci), lambda j: (0, j, 0)),
            pl.BlockSpec((B, KBLK, UNITS), lambda j: (0, j, 0)),
            pl.BlockSpec((KBLK, 1), lambda j: (j, 0)),
            pl.BlockSpec((N, 1), lambda j: (0, 0)),
        ],
        out_specs=[
            pl.BlockSpec((KBLK, F), lambda j: (j, 0)),
            pl.BlockSpec((N, F), lambda j: (0, 0)),
        ],
        out_shape=[
            jax.ShapeDtypeStruct((N, F), jnp.float32),
            jax.ShapeDtypeStruct((N, F), jnp.float32),
        ],
    )(amax, inp_b, h_b, dis, dis)


def _sapply2_body(a_ref, x_ref, disj_ref, disi_ref, z_ref, out_ref):
    j = pl.program_id(0)
    xs = (x_ref[...] * disj_ref[...]).astype(jnp.bfloat16)
    part = jnp.dot(a_ref[...], xs, preferred_element_type=jnp.float32)

    @pl.when(j == 0)
    def _():
        out_ref[...] = part

    @pl.when(j > 0)
    def _():
        out_ref[...] += part

    @pl.when(j == NJ - 1)
    def _():
        out_ref[...] = -2.0 * disi_ref[...] * out_ref[...] - z_ref[...]


def _sapply2(amax, dis, x1, x0):
    return pl.pallas_call(
        _sapply2_body,
        grid=(NJ,),
        in_specs=[
            pl.BlockSpec((N, KBLK), lambda j: (0, j)),
            pl.BlockSpec((KBLK, F), lambda j: (j, 0)),
            pl.BlockSpec((KBLK, 1), lambda j: (j, 0)),
            pl.BlockSpec((N, 1), lambda j: (0, 0)),
            pl.BlockSpec((N, F), lambda j: (0, 0)),
        ],
        out_specs=pl.BlockSpec((N, F), lambda j: (0, 0)),
        out_shape=jax.ShapeDtypeStruct((N, F), jnp.float32),
    )(amax, x1, dis, dis, x0)


def _gate_body(x0_ref, x1_ref, x2_ref, wr_ref, wu_ref, br_ref, bu_ref,
               hx_ref, rh_ref, u_ref):
    xs = (x0_ref[...], x1_ref[...], x2_ref[...])
    for b in range(B):
        accr = br_ref[...]
        accu = bu_ref[...]
        for m in range(M):
            xb = xs[m][:, b * CPAD:(b + 1) * CPAD]
            accr = accr + jnp.dot(xb, wr_ref[m],
                                  preferred_element_type=jnp.float32)
            accu = accu + jnp.dot(xb, wu_ref[m],
                                  preferred_element_type=jnp.float32)
        r = jax.nn.sigmoid(accr)
        rh_ref[b] = r * hx_ref[b]
        u_ref[b] = jax.nn.sigmoid(accu)


def _gate(x0, x1, x2, wr, wu, br, bu, hx_b):
    return pl.pallas_call(
        _gate_body,
        grid=(NRB,),
        in_specs=[
            pl.BlockSpec((RBLK, F), lambda i: (i, 0)),
            pl.BlockSpec((RBLK, F), lambda i: (i, 0)),
            pl.BlockSpec((RBLK, F), lambda i: (i, 0)),
            pl.BlockSpec((M, CPAD, UNITS), lambda i: (0, 0, 0)),
            pl.BlockSpec((M, CPAD, UNITS), lambda i: (0, 0, 0)),
            pl.BlockSpec((1, UNITS), lambda i: (0, 0)),
            pl.BlockSpec((1, UNITS), lambda i: (0, 0)),
            pl.BlockSpec((B, RBLK, UNITS), lambda i: (0, i, 0)),
        ],
        out_specs=[
            pl.BlockSpec((B, RBLK, UNITS), lambda i: (0, i, 0)),
            pl.BlockSpec((B, RBLK, UNITS), lambda i: (0, i, 0)),
        ],
        out_shape=[
            jax.ShapeDtypeStruct((B, N, UNITS), jnp.float32),
            jax.ShapeDtypeStruct((B, N, UNITS), jnp.float32),
        ],
    )(x0, x1, x2, wr, wu, br, bu, hx_b)


def _cand_body(x0_ref, x1_ref, x2_ref, wc_ref, bc_ref, u_ref, hx_ref,
               nh_ref):
    xs = (x0_ref[...], x1_ref[...], x2_ref[...])
    for b in range(B):
        acc = bc_ref[...]
        for m in range(M):
            xb = xs[m][:, b * CPAD:(b + 1) * CPAD]
            acc = acc + jnp.dot(xb, wc_ref[m],
                                preferred_element_type=jnp.float32)
        c = jnp.tanh(acc)
        u = u_ref[b]
        nh_ref[b] = u * hx_ref[b] + (1.0 - u) * c


def _cand(x0, x1, x2, wc, bc, u_b, hx_b):
    return pl.pallas_call(
        _cand_body,
        grid=(NRB,),
        in_specs=[
            pl.BlockSpec((RBLK, F), lambda i: (i, 0)),
            pl.BlockSpec((RBLK, F), lambda i: (i, 0)),
            pl.BlockSpec((RBLK, F), lambda i: (i, 0)),
            pl.BlockSpec((M, CPAD, UNITS), lambda i: (0, 0, 0)),
            pl.BlockSpec((1, UNITS), lambda i: (0, 0)),
            pl.BlockSpec((B, RBLK, UNITS), lambda i: (0, i, 0)),
            pl.BlockSpec((B, RBLK, UNITS), lambda i: (0, i, 0)),
        ],
        out_specs=pl.BlockSpec((B, RBLK, UNITS), lambda i: (0, i, 0)),
        out_shape=jax.ShapeDtypeStruct((B, N, UNITS), jnp.float32),
    )(x0, x1, x2, wc, bc, u_b, hx_b)


def _prep_w(W, C, O):
    Wr = jnp.transpose(W.reshape(C, M, O), (1, 0, 2))
    return jnp.pad(Wr, ((0, 0), (0, CPAD - C), (0, 0)))


def kernel(inputs, hidden_state, adj, W0_gate, b0_gate, W0_cand, b0_cand,
           W1_gate, b1_gate, W1_cand, b1_cand):
    amax, dis = _build(adj)

    params = [(W0_gate, b0_gate, W0_cand, b0_cand, 1),
              (W1_gate, b1_gate, W1_cand, b1_cand, UNITS)]
    cur = inputs.reshape(B, N, 1)
    hs = []
    for l in range(2):
        Wg, bg, Wc, bc, ci = params[l]
        C = ci + UNITS
        wg = _prep_w(Wg, C, 2 * UNITS)
        wr, wu = wg[:, :, :UNITS], wg[:, :, UNITS:]
        br = bg[:UNITS].reshape(1, UNITS)
        bu = bg[UNITS:].reshape(1, UNITS)
        wc = _prep_w(Wc, C, UNITS)
        bcv = bc.reshape(1, UNITS)
        hx_b = hidden_state[l].reshape(B, N, UNITS)

        x0g, x1g = _sapply1(amax, dis, cur, hx_b, ci)
        x2g = _sapply2(amax, dis, x1g, x0g)
        rh_b, u_b = _gate(x0g, x1g, x2g, wr, wu, br, bu, hx_b)

        x0c, x1c = _sapply1(amax, dis, cur, rh_b, ci)
        x2c = _sapply2(amax, dis, x1c, x0c)
        nh_b = _cand(x0c, x1c, x2c, wc, bcv, u_b, hx_b)

        hs.append(nh_b.reshape(B, N * UNITS))
        cur = nh_b

    return hs[-1], jnp.stack(hs, axis=0)
